# Initial kernel scaffold; baseline (speedup 1.0000x reference)
import jax
import jax.numpy as jnp
from jax.experimental import pallas as pl


def kernel(weights, types, system_indices, type_to_index):
    raise NotImplementedError("write your pallas kernel here")


# speedup vs baseline: 88.7013x; 88.7013x over previous
import functools

import jax
import jax.numpy as jnp
from jax import lax
from jax.experimental import pallas as pl
from jax.experimental.pallas import tpu as pltpu
from jax.experimental.pallas import tpu_sc as plsc

_N_ATOMS = 1_000_000
_N_TYPES = 119
_N_PROPS = 128
_N_SYSTEMS = 1024

_NW = 32
_CH = 32_000
_CHUNK = 1_600
_VPC = _CHUNK // 16


def _sc_hist_body(types_hbm, sys_hbm, t2i_hbm, zeros_hbm, counts_hbm,
                  t2i_v, table_v, types_v, sys_v):
  c = lax.axis_index("c")
  s = lax.axis_index("s")
  wid = s * 2 + c
  base = wid * _CH
  nchunks = jnp.minimum(_CH, _N_ATOMS - base) // _CHUNK

  pltpu.sync_copy(t2i_hbm, t2i_v)
  pltpu.sync_copy(zeros_hbm, table_v)
  ones = jnp.full((16,), 1.0, jnp.float32)

  def chunk_body(ci, carry):
    off = pl.multiple_of(base + ci * _CHUNK, 8)
    pltpu.sync_copy(types_hbm.at[pl.ds(off, _CHUNK)], types_v)
    pltpu.sync_copy(sys_hbm.at[pl.ds(off, _CHUNK)], sys_v)

    def vec_body(vi, carry2):
      t = types_v[pl.ds(vi * 16, 16)]
      sy = sys_v[pl.ds(vi * 16, 16)]
      row = plsc.load_gather(t2i_v, [t])
      plsc.addupdate_scatter(table_v, [sy, row], ones)
      return carry2

    lax.fori_loop(0, _VPC, vec_body, 0)
    return carry

  lax.fori_loop(0, nchunks, chunk_body, 0)
  pltpu.sync_copy(table_v, counts_hbm.at[wid])


_sc_hist = pl.kernel(
    _sc_hist_body,
    out_type=jax.ShapeDtypeStruct((_NW, _N_SYSTEMS, _N_TYPES), jnp.float32),
    mesh=plsc.VectorSubcoreMesh(core_axis_name="c", subcore_axis_name="s"),
    scratch_types=[
        pltpu.VMEM((128,), jnp.int32),
        pltpu.VMEM((_N_SYSTEMS, _N_TYPES), jnp.float32),
        pltpu.VMEM((_CHUNK,), jnp.int32),
        pltpu.VMEM((_CHUNK,), jnp.int32),
    ],
    compiler_params=pltpu.CompilerParams(
        needs_layout_passes=False, use_tc_tiling_on_sc=False),
)


def _tc_body(counts_ref, w_ref, out_ref):
  c = jnp.sum(counts_ref[...], axis=0)
  out_ref[...] = jnp.dot(c, w_ref[...], preferred_element_type=jnp.float32,
                         precision=lax.Precision.HIGHEST)


_SYS_BLK = 128


def _tc_reduce_matmul(counts, weights):
  grid = _N_SYSTEMS // _SYS_BLK
  return pl.pallas_call(
      _tc_body,
      grid=(grid,),
      in_specs=[
          pl.BlockSpec((_NW, _SYS_BLK, _N_TYPES), lambda i: (0, i, 0)),
          pl.BlockSpec((_N_TYPES, _N_PROPS), lambda i: (0, 0)),
      ],
      out_specs=pl.BlockSpec((_SYS_BLK, _N_PROPS), lambda i: (i, 0)),
      out_shape=jax.ShapeDtypeStruct((_N_SYSTEMS, _N_PROPS), jnp.float32),
  )(counts, weights)


@jax.jit
def kernel(weights, types, system_indices, type_to_index):
  t2i_pad = jnp.zeros((128,), jnp.int32).at[:_N_TYPES].set(type_to_index)
  zeros = jnp.zeros((_N_SYSTEMS, _N_TYPES), jnp.float32)
  counts = _sc_hist(types, system_indices, t2i_pad, zeros)
  return _tc_reduce_matmul(counts, weights)


# speedup vs baseline: 111.9214x; 1.2618x over previous
import functools

import jax
import jax.numpy as jnp
from jax import lax
from jax.experimental import pallas as pl
from jax.experimental.pallas import tpu as pltpu
from jax.experimental.pallas import tpu_sc as plsc

_N_ATOMS = 1_000_000
_N_TYPES = 119
_N_PROPS = 128
_N_SYSTEMS = 1024

_NW = 32
_CH = 32_000
_CHUNK = 2_000
_VPC = _CHUNK // 16
_UNROLL = 5


def _sc_hist_body(types_hbm, sys_hbm, t2i_hbm, zeros_hbm, counts_hbm,
                  t2i_v, table_v, types_v, sys_v, tsem, ssem):
  c = lax.axis_index("c")
  s = lax.axis_index("s")
  wid = s * 2 + c
  base = wid * _CH
  nchunks = jnp.minimum(_CH, _N_ATOMS - base) // _CHUNK

  def start_load(ci, buf):
    off = pl.multiple_of(base + ci * _CHUNK, 8)
    pltpu.async_copy(types_hbm.at[pl.ds(off, _CHUNK)], types_v.at[buf],
                     tsem.at[buf])
    pltpu.async_copy(sys_hbm.at[pl.ds(off, _CHUNK)], sys_v.at[buf],
                     ssem.at[buf])

  def wait_load(ci, buf):
    off = pl.multiple_of(base + ci * _CHUNK, 8)
    pltpu.make_async_copy(types_hbm.at[pl.ds(off, _CHUNK)], types_v.at[buf],
                          tsem.at[buf]).wait()
    pltpu.make_async_copy(sys_hbm.at[pl.ds(off, _CHUNK)], sys_v.at[buf],
                          ssem.at[buf]).wait()

  pltpu.sync_copy(t2i_hbm, t2i_v)
  start_load(0, 0)
  pltpu.sync_copy(zeros_hbm, table_v)
  ones = jnp.full((16,), 1.0, jnp.float32)

  def chunk_body(ci, carry):
    buf = lax.rem(ci, 2)

    @pl.when(ci + 1 < nchunks)
    def _():
      start_load(ci + 1, 1 - buf)

    wait_load(ci, buf)

    def vec_body(gi, carry2):
      for j in range(_UNROLL):
        vi = gi * _UNROLL + j
        t = types_v[buf, pl.ds(vi * 16, 16)]
        sy = sys_v[buf, pl.ds(vi * 16, 16)]
        row = plsc.load_gather(t2i_v, [t])
        plsc.addupdate_scatter(table_v, [sy, row], ones)
      return carry2

    lax.fori_loop(0, _VPC // _UNROLL, vec_body, 0)
    return carry

  lax.fori_loop(0, nchunks, chunk_body, 0)
  pltpu.sync_copy(table_v, counts_hbm.at[wid])


_sc_hist = pl.kernel(
    _sc_hist_body,
    out_type=jax.ShapeDtypeStruct((_NW, _N_SYSTEMS, _N_TYPES), jnp.float32),
    mesh=plsc.VectorSubcoreMesh(core_axis_name="c", subcore_axis_name="s"),
    scratch_types=[
        pltpu.VMEM((128,), jnp.int32),
        pltpu.VMEM((_N_SYSTEMS, _N_TYPES), jnp.float32),
        pltpu.VMEM((2, _CHUNK), jnp.int32),
        pltpu.VMEM((2, _CHUNK), jnp.int32),
        pltpu.SemaphoreType.DMA((2,)),
        pltpu.SemaphoreType.DMA((2,)),
    ],
    compiler_params=pltpu.CompilerParams(
        needs_layout_passes=False, use_tc_tiling_on_sc=False),
)


def _tc_body(counts_ref, w_ref, out_ref):
  c = jnp.sum(counts_ref[...], axis=0)
  out_ref[...] = jnp.dot(c, w_ref[...], preferred_element_type=jnp.float32,
                         precision=lax.Precision.HIGHEST)


_SYS_BLK = 128


def _tc_reduce_matmul(counts, weights):
  grid = _N_SYSTEMS // _SYS_BLK
  return pl.pallas_call(
      _tc_body,
      grid=(grid,),
      in_specs=[
          pl.BlockSpec((_NW, _SYS_BLK, _N_TYPES), lambda i: (0, i, 0)),
          pl.BlockSpec((_N_TYPES, _N_PROPS), lambda i: (0, 0)),
      ],
      out_specs=pl.BlockSpec((_SYS_BLK, _N_PROPS), lambda i: (i, 0)),
      out_shape=jax.ShapeDtypeStruct((_N_SYSTEMS, _N_PROPS), jnp.float32),
  )(counts, weights)


@jax.jit
def kernel(weights, types, system_indices, type_to_index):
  t2i_pad = jnp.zeros((128,), jnp.int32).at[:_N_TYPES].set(type_to_index)
  zeros = jnp.zeros((_N_SYSTEMS, _N_TYPES), jnp.float32)
  counts = _sc_hist(types, system_indices, t2i_pad, zeros)
  return _tc_reduce_matmul(counts, weights)


# speedup vs baseline: 137.2877x; 1.2266x over previous
import functools

import jax
import jax.numpy as jnp
from jax import lax
from jax.experimental import pallas as pl
from jax.experimental.pallas import tpu as pltpu
from jax.experimental.pallas import tpu_sc as plsc

_N_ATOMS = 1_000_000
_N_TYPES = 119
_N_PROPS = 128
_N_SYSTEMS = 1024

_NW = 32
_CH = 32_000
_CHUNK = 2_000
_VPC = _CHUNK // 16
_UNROLL = 5


def _sc_hist_body(types_hbm, sys_hbm, t2i_hbm, zeros_hbm, counts_hbm,
                  t2i_v, table_v, types_v, sys_v, tsem, ssem):
  c = lax.axis_index("c")
  s = lax.axis_index("s")
  wid = s * 2 + c
  base = wid * _CH
  nchunks = jnp.minimum(_CH, _N_ATOMS - base) // _CHUNK

  def start_load(ci, buf):
    off = pl.multiple_of(base + ci * _CHUNK, 8)
    pltpu.async_copy(types_hbm.at[pl.ds(off, _CHUNK)], types_v.at[buf],
                     tsem.at[buf])
    pltpu.async_copy(sys_hbm.at[pl.ds(off, _CHUNK)], sys_v.at[buf],
                     ssem.at[buf])

  def wait_load(ci, buf):
    off = pl.multiple_of(base + ci * _CHUNK, 8)
    pltpu.make_async_copy(types_hbm.at[pl.ds(off, _CHUNK)], types_v.at[buf],
                          tsem.at[buf]).wait()
    pltpu.make_async_copy(sys_hbm.at[pl.ds(off, _CHUNK)], sys_v.at[buf],
                          ssem.at[buf]).wait()

  pltpu.sync_copy(t2i_hbm, t2i_v.at[pl.ds(0, _N_TYPES)])
  start_load(0, 0)
  pltpu.sync_copy(zeros_hbm, table_v)
  ones = jnp.full((16,), 1.0, jnp.float32)

  def chunk_body(ci, carry):
    buf = lax.rem(ci, 2)

    @pl.when(ci + 1 < nchunks)
    def _():
      start_load(ci + 1, 1 - buf)

    wait_load(ci, buf)

    def vec_body(vi):
      t = types_v[buf, pl.ds(vi * 16, 16)]
      sy = sys_v[buf, pl.ds(vi * 16, 16)]
      row = plsc.load_gather(t2i_v, [t])
      plsc.addupdate_scatter(table_v, [sy, row], ones)

    plsc.parallel_loop(0, _VPC, 1, unroll=_UNROLL)(vec_body)
    return carry

  lax.fori_loop(0, nchunks, chunk_body, 0)
  pltpu.sync_copy(table_v, counts_hbm.at[wid])


_sc_hist = pl.kernel(
    _sc_hist_body,
    out_type=jax.ShapeDtypeStruct((_NW, _N_SYSTEMS, _N_TYPES), jnp.float32),
    mesh=plsc.VectorSubcoreMesh(core_axis_name="c", subcore_axis_name="s"),
    scratch_types=[
        pltpu.VMEM((128,), jnp.int32),
        pltpu.VMEM((_N_SYSTEMS, _N_TYPES), jnp.float32),
        pltpu.VMEM((2, _CHUNK), jnp.int32),
        pltpu.VMEM((2, _CHUNK), jnp.int32),
        pltpu.SemaphoreType.DMA((2,)),
        pltpu.SemaphoreType.DMA((2,)),
    ],
    compiler_params=pltpu.CompilerParams(
        needs_layout_passes=False, use_tc_tiling_on_sc=False),
)


def _tc_body(counts_ref, w_ref, out_ref):
  c = jnp.sum(counts_ref[...], axis=0)
  out_ref[...] = jnp.dot(c, w_ref[...], preferred_element_type=jnp.float32,
                         precision=lax.Precision.HIGHEST)


_SYS_BLK = 128


def _tc_reduce_matmul(counts, weights):
  grid = _N_SYSTEMS // _SYS_BLK
  return pl.pallas_call(
      _tc_body,
      grid=(grid,),
      in_specs=[
          pl.BlockSpec((_NW, _SYS_BLK, _N_TYPES), lambda i: (0, i, 0)),
          pl.BlockSpec((_N_TYPES, _N_PROPS), lambda i: (0, 0)),
      ],
      out_specs=pl.BlockSpec((_SYS_BLK, _N_PROPS), lambda i: (i, 0)),
      out_shape=jax.ShapeDtypeStruct((_N_SYSTEMS, _N_PROPS), jnp.float32),
  )(counts, weights)


@jax.jit
def kernel(weights, types, system_indices, type_to_index):
  zeros = jnp.zeros((_N_SYSTEMS, _N_TYPES), jnp.float32)
  counts = _sc_hist(types, system_indices, type_to_index, zeros)
  return _tc_reduce_matmul(counts, weights)
